# initial kernel scaffold (unmeasured)
import jax
import jax.numpy as jnp
from jax import lax
from jax.experimental import pallas as pl
from jax.experimental.pallas import tpu as pltpu

N_DEV = 4
SQ = 2048
D = 1024
H = 8
DH = 128
G = 4
GR = SQ // G
SCALE = 0.08838834764831843


def _group(a):
    return a.reshape(G * 2, G, 64, a.shape[-1]).transpose(1, 0, 2, 3).reshape(
        SQ, a.shape[-1]
    )


def _ungroup(a):
    return a.reshape(G, G * 2, 64, a.shape[-1]).transpose(1, 0, 2, 3).reshape(
        SQ, a.shape[-1]
    )


def _body(xg_ref, wq_ref, kg_ref, vg_ref, wo_ref, out_ref,
          kcomm, vcomm, ctx_ref,
          ksend, krecv, vsend, vrecv):
    my = lax.axis_index("i")
    right = (my + 1) % N_DEV
    left = (my - 1) % N_DEV

    barrier_sem = pltpu.get_barrier_semaphore()
    for nbr in (left, right):
        pl.semaphore_signal(
            barrier_sem, inc=1,
            device_id=(nbr,), device_id_type=pl.DeviceIdType.MESH,
        )
    pl.semaphore_wait(barrier_sem, 2)

    for h in range(N_DEV - 1):
        k_src = kg_ref if h == 0 else kcomm.at[h - 1]
        v_src = vg_ref if h == 0 else vcomm.at[h - 1]
        k_rdma = pltpu.make_async_remote_copy(
            src_ref=k_src, dst_ref=kcomm.at[h],
            send_sem=ksend.at[h], recv_sem=krecv.at[h],
            device_id=(right,), device_id_type=pl.DeviceIdType.MESH,
        )
        v_rdma = pltpu.make_async_remote_copy(
            src_ref=v_src, dst_ref=vcomm.at[h],
            send_sem=vsend.at[h], recv_sem=vrecv.at[h],
            device_id=(right,), device_id_type=pl.DeviceIdType.MESH,
        )
        k_rdma.start()
        v_rdma.start()
        k_rdma.wait()
        v_rdma.wait()

    for r in range(G):
        rs = r * GR
        for h in range(H):
            hs = h * DH
            q = lax.dot_general(
                xg_ref[rs:rs + GR, :], wq_ref[:, hs:hs + DH],
                (((1,), (0,)), ((), ())),
                preferred_element_type=jnp.float32,
            )
            q = (q * SCALE).astype(jnp.bfloat16)
            k_all = jnp.concatenate(
                [kg_ref[rs:rs + GR, hs:hs + DH]]
                + [kcomm[s, rs:rs + GR, hs:hs + DH] for s in range(3)],
                axis=0,
            )
            s = lax.dot_general(
                q, k_all, (((1,), (1,)), ((), ())),
                preferred_element_type=jnp.float32,
            )
            m = jnp.max(s, axis=1, keepdims=True)
            p = jnp.exp(s - m)
            denom = jnp.sum(p, axis=1, keepdims=True)
            p = (p / denom).astype(jnp.bfloat16)
            v_all = jnp.concatenate(
                [vg_ref[rs:rs + GR, hs:hs + DH]]
                + [vcomm[s2, rs:rs + GR, hs:hs + DH] for s2 in range(3)],
                axis=0,
            )
            ctx = lax.dot_general(
                p, v_all, (((1,), (0,)), ((), ())),
                preferred_element_type=jnp.float32,
            )
            ctx_ref[rs:rs + GR, hs:hs + DH] = ctx.astype(jnp.bfloat16)

    for t in range(G):
        ts = t * GR
        out_ref[ts:ts + GR, :] = lax.dot_general(
            ctx_ref[ts:ts + GR, :], wo_ref[:, :],
            (((1,), (0,)), ((), ())),
            preferred_element_type=jnp.float32,
        ).astype(jnp.bfloat16)


def kernel(x, Wq, K_ext, V_ext, Wo):
    xg = _group(x[0].astype(jnp.bfloat16))
    kg = _group(K_ext[0].reshape(SQ, D).astype(jnp.bfloat16))
    vg = _group(V_ext[0].reshape(SQ, D).astype(jnp.bfloat16))
    wq = Wq.astype(jnp.bfloat16)
    wo = Wo.astype(jnp.bfloat16)

    outg = pl.pallas_call(
        _body,
        out_shape=jax.ShapeDtypeStruct((SQ, D), jnp.bfloat16),
        in_specs=[pl.BlockSpec(memory_space=pltpu.VMEM)] * 5,
        out_specs=pl.BlockSpec(memory_space=pltpu.VMEM),
        scratch_shapes=[
            pltpu.VMEM((N_DEV - 1, SQ, D), jnp.bfloat16),
            pltpu.VMEM((N_DEV - 1, SQ, D), jnp.bfloat16),
            pltpu.VMEM((SQ, D), jnp.bfloat16),
            pltpu.SemaphoreType.DMA((N_DEV - 1,)),
            pltpu.SemaphoreType.DMA((N_DEV - 1,)),
            pltpu.SemaphoreType.DMA((N_DEV - 1,)),
            pltpu.SemaphoreType.DMA((N_DEV - 1,)),
        ],
        compiler_params=pltpu.CompilerParams(collective_id=0),
    )(xg, wq, kg, vg, wo)

    out = _ungroup(outg)
    return out.astype(jnp.float32)[None]


# baseline (device time: 418646 ns/iter reference)
import jax
import jax.numpy as jnp
from jax import lax
from jax.experimental import pallas as pl
from jax.experimental.pallas import tpu as pltpu

N_DEV = 4
SQ = 2048
D = 1024
H = 8
DH = 128
G = 4
GR = SQ // G
SCALE = 0.08838834764831843


def _group(a):
    return a.reshape(G * 2, G, 64, a.shape[-1]).transpose(1, 0, 2, 3).reshape(
        SQ, a.shape[-1]
    )


def _ungroup(a):
    return a.reshape(G, G * 2, 64, a.shape[-1]).transpose(1, 0, 2, 3).reshape(
        SQ, a.shape[-1]
    )


def _body(xg_ref, wq_ref, kg_ref, vg_ref, wo_ref, out_ref,
          kcomm, vcomm,
          ksend, krecv, vsend, vrecv):
    my = lax.axis_index("i")
    right = (my + 1) % N_DEV
    left = (my - 1) % N_DEV

    barrier_sem = pltpu.get_barrier_semaphore()
    for nbr in (left, right):
        pl.semaphore_signal(
            barrier_sem, inc=1,
            device_id=(nbr,), device_id_type=pl.DeviceIdType.MESH,
        )
    pl.semaphore_wait(barrier_sem, 2)

    for h in range(N_DEV - 1):
        k_src = kg_ref if h == 0 else kcomm.at[h - 1]
        v_src = vg_ref if h == 0 else vcomm.at[h - 1]
        k_rdma = pltpu.make_async_remote_copy(
            src_ref=k_src, dst_ref=kcomm.at[h],
            send_sem=ksend.at[h], recv_sem=krecv.at[h],
            device_id=(right,), device_id_type=pl.DeviceIdType.MESH,
        )
        v_rdma = pltpu.make_async_remote_copy(
            src_ref=v_src, dst_ref=vcomm.at[h],
            send_sem=vsend.at[h], recv_sem=vrecv.at[h],
            device_id=(right,), device_id_type=pl.DeviceIdType.MESH,
        )
        k_rdma.start()
        v_rdma.start()
        k_rdma.wait()
        v_rdma.wait()

    def attn_step(idx, carry):
        rs = (idx // H) * GR
        hs = (idx % H) * DH
        q = lax.dot_general(
            xg_ref[pl.ds(rs, GR), :], wq_ref[:, pl.ds(hs, DH)],
            (((1,), (0,)), ((), ())),
            preferred_element_type=jnp.float32,
        )
        q = (q * SCALE).astype(jnp.bfloat16)
        k_all = jnp.concatenate(
            [kg_ref[pl.ds(rs, GR), pl.ds(hs, DH)]]
            + [kcomm[s, pl.ds(rs, GR), pl.ds(hs, DH)] for s in range(3)],
            axis=0,
        )
        s = lax.dot_general(
            q, k_all, (((1,), (1,)), ((), ())),
            preferred_element_type=jnp.float32,
        )
        m = jnp.max(s, axis=1, keepdims=True)
        p = jnp.exp(s - m)
        denom = jnp.sum(p, axis=1, keepdims=True)
        p = (p / denom).astype(jnp.bfloat16)
        v_all = jnp.concatenate(
            [vg_ref[pl.ds(rs, GR), pl.ds(hs, DH)]]
            + [vcomm[s2, pl.ds(rs, GR), pl.ds(hs, DH)] for s2 in range(3)],
            axis=0,
        )
        ctx = lax.dot_general(
            p, v_all, (((1,), (0,)), ((), ())),
            preferred_element_type=jnp.float32,
        )
        out_ref[pl.ds(rs, GR), pl.ds(hs, DH)] = ctx.astype(jnp.bfloat16)
        return carry

    lax.fori_loop(0, G * H, attn_step, 0)

    def proj_step(t, carry):
        ts = t * GR
        ctx_tile = out_ref[pl.ds(ts, GR), :]
        out_ref[pl.ds(ts, GR), :] = lax.dot_general(
            ctx_tile, wo_ref[:, :],
            (((1,), (0,)), ((), ())),
            preferred_element_type=jnp.float32,
        ).astype(jnp.bfloat16)
        return carry

    lax.fori_loop(0, G, proj_step, 0)


def kernel(x, Wq, K_ext, V_ext, Wo):
    xg = _group(x[0].astype(jnp.bfloat16))
    kg = _group(K_ext[0].reshape(SQ, D).astype(jnp.bfloat16))
    vg = _group(V_ext[0].reshape(SQ, D).astype(jnp.bfloat16))
    wq = Wq.astype(jnp.bfloat16)
    wo = Wo.astype(jnp.bfloat16)

    outg = pl.pallas_call(
        _body,
        out_shape=jax.ShapeDtypeStruct((SQ, D), jnp.bfloat16),
        in_specs=[pl.BlockSpec(memory_space=pltpu.VMEM)] * 5,
        out_specs=pl.BlockSpec(memory_space=pltpu.VMEM),
        scratch_shapes=[
            pltpu.VMEM((N_DEV - 1, SQ, D), jnp.bfloat16),
            pltpu.VMEM((N_DEV - 1, SQ, D), jnp.bfloat16),
            pltpu.SemaphoreType.DMA((N_DEV - 1,)),
            pltpu.SemaphoreType.DMA((N_DEV - 1,)),
            pltpu.SemaphoreType.DMA((N_DEV - 1,)),
            pltpu.SemaphoreType.DMA((N_DEV - 1,)),
        ],
        compiler_params=pltpu.CompilerParams(
            collective_id=0,
            vmem_limit_bytes=44 * 1024 * 1024,
        ),
    )(xg, wq, kg, vg, wo)

    out = _ungroup(outg)
    return out.astype(jnp.float32)[None]


# device time: 223148 ns/iter; 1.8761x vs baseline; 1.8761x over previous
import jax
import jax.numpy as jnp
from jax import lax
from jax.experimental import pallas as pl
from jax.experimental.pallas import tpu as pltpu

N_DEV = 4
SQ = 2048
HSQ = SQ // 2
D = 1024
H = 8
DH = 128
G = 4
GR = SQ // G
SCALE = 0.08838834764831843


def _group(a):
    return a.reshape(G * 2, G, 64, a.shape[-1]).transpose(1, 0, 2, 3).reshape(
        SQ, a.shape[-1]
    )


def _ungroup(a):
    return a.reshape(G, G * 2, 64, a.shape[-1]).transpose(1, 0, 2, 3).reshape(
        SQ, a.shape[-1]
    )


def _body(xg_ref, wq_ref, kg_ref, vg_ref, wo_ref, out_ref,
          kcomm, vcomm, acc_ref, l_ref,
          ksend, krecv, vsend, vrecv):
    q_ref = out_ref
    my = lax.axis_index("i")
    right = (my + 1) % N_DEV
    left = (my - 1) % N_DEV

    barrier_sem = pltpu.get_barrier_semaphore()
    for nbr in (left, right):
        pl.semaphore_signal(
            barrier_sem, inc=1,
            device_id=(nbr,), device_id_type=pl.DeviceIdType.MESH,
        )
    pl.semaphore_wait(barrier_sem, 2)

    def rdma(src, dst, ssem, rsem, dev):
        return pltpu.make_async_remote_copy(
            src_ref=src, dst_ref=dst, send_sem=ssem, recv_sem=rsem,
            device_id=(dev,), device_id_type=pl.DeviceIdType.MESH,
        )

    k1r = rdma(kg_ref, kcomm.at[0], ksend.at[0], krecv.at[0], right)
    k1l = rdma(kg_ref, kcomm.at[1], ksend.at[1], krecv.at[1], left)
    v1r = rdma(vg_ref, vcomm.at[0], vsend.at[0], vrecv.at[0], right)
    v1l = rdma(vg_ref, vcomm.at[1], vsend.at[1], vrecv.at[1], left)
    k1r.start()
    v1r.start()
    k1l.start()
    v1l.start()

    k2r = rdma(kcomm.at[0, pl.ds(0, HSQ)], kcomm.at[2, pl.ds(0, HSQ)],
               ksend.at[2], krecv.at[2], right)
    v2r = rdma(vcomm.at[0, pl.ds(0, HSQ)], vcomm.at[2, pl.ds(0, HSQ)],
               vsend.at[2], vrecv.at[2], right)
    k2l = rdma(kcomm.at[1, pl.ds(HSQ, HSQ)], kcomm.at[2, pl.ds(HSQ, HSQ)],
               ksend.at[3], krecv.at[3], left)
    v2l = rdma(vcomm.at[1, pl.ds(HSQ, HSQ)], vcomm.at[2, pl.ds(HSQ, HSQ)],
               vsend.at[3], vrecv.at[3], left)

    QT = GR // 2

    def q_step(t, carry):
        ts = t * QT
        q = lax.dot_general(
            xg_ref[pl.ds(ts, QT), :], wq_ref[:, :],
            (((1,), (0,)), ((), ())),
            preferred_element_type=jnp.float32,
        )
        q_ref[pl.ds(ts, QT), :] = (q * SCALE).astype(jnp.bfloat16)
        return carry

    lax.fori_loop(0, SQ // QT, q_step, 0)

    def chunk_pass(k_chunk, v_chunk, first):
        for h in range(H):
            hs = h * DH

            def step(t, carry, hs=hs, h=h):
                rs = t * QT
                ks = (t // 2) * GR
                q = q_ref[pl.ds(rs, QT), pl.ds(hs, DH)]
                k = k_chunk[pl.ds(ks, GR), pl.ds(hs, DH)]
                s = lax.dot_general(
                    q, k, (((1,), (1,)), ((), ())),
                    preferred_element_type=jnp.float32,
                )
                p = jnp.exp(s)
                lsum = jnp.sum(p, axis=1, keepdims=True)
                pv = lax.dot_general(
                    p.astype(jnp.bfloat16),
                    v_chunk[pl.ds(ks, GR), pl.ds(hs, DH)],
                    (((1,), (0,)), ((), ())),
                    preferred_element_type=jnp.float32,
                )
                if first:
                    acc_ref[pl.ds(rs, QT), pl.ds(hs, DH)] = pv.astype(jnp.bfloat16)
                    l_ref[pl.ds(rs, QT), h:h + 1] = lsum
                else:
                    acc_ref[pl.ds(rs, QT), pl.ds(hs, DH)] += pv.astype(jnp.bfloat16)
                    l_ref[pl.ds(rs, QT), h:h + 1] += lsum
                return carry

            lax.fori_loop(0, SQ // QT, step, 0)

    chunk_pass(kg_ref, vg_ref, first=True)

    k1r.wait_recv()
    v1r.wait_recv()
    k2r.start()
    v2r.start()
    k1l.wait_recv()
    v1l.wait_recv()
    k2l.start()
    v2l.start()

    chunk_pass(kcomm.at[0], vcomm.at[0], first=False)
    chunk_pass(kcomm.at[1], vcomm.at[1], first=False)

    k2r.wait_recv()
    v2r.wait_recv()
    k2l.wait_recv()
    v2l.wait_recv()
    chunk_pass(kcomm.at[2], vcomm.at[2], first=False)

    for h in range(H):
        hs = h * DH

        def norm_step(t, carry, hs=hs, h=h):
            ts = t * QT
            rinv = 1.0 / l_ref[pl.ds(ts, QT), h:h + 1]
            out_ref[pl.ds(ts, QT), pl.ds(hs, DH)] = (
                acc_ref[pl.ds(ts, QT), pl.ds(hs, DH)] * rinv
            ).astype(jnp.bfloat16)
            return carry

        lax.fori_loop(0, SQ // QT, norm_step, 0)

    def proj_step(t, carry):
        ts = t * QT
        ctx_tile = out_ref[pl.ds(ts, QT), :]
        out_ref[pl.ds(ts, QT), :] = lax.dot_general(
            ctx_tile, wo_ref[:, :],
            (((1,), (0,)), ((), ())),
            preferred_element_type=jnp.float32,
        ).astype(jnp.bfloat16)
        return carry

    lax.fori_loop(0, SQ // QT, proj_step, 0)

    for d in (k1r, v1r, k1l, v1l, k2r, v2r, k2l, v2l):
        d.wait_send()


def kernel(x, Wq, K_ext, V_ext, Wo):
    xg = _group(x[0].astype(jnp.bfloat16))
    kg = _group(K_ext[0].reshape(SQ, D).astype(jnp.bfloat16))
    vg = _group(V_ext[0].reshape(SQ, D).astype(jnp.bfloat16))
    wq = Wq.astype(jnp.bfloat16)
    wo = Wo.astype(jnp.bfloat16)

    outg = pl.pallas_call(
        _body,
        out_shape=jax.ShapeDtypeStruct((SQ, D), jnp.bfloat16),
        in_specs=[pl.BlockSpec(memory_space=pltpu.VMEM)] * 5,
        out_specs=pl.BlockSpec(memory_space=pltpu.VMEM),
        scratch_shapes=[
            pltpu.VMEM((3, SQ, D), jnp.bfloat16),
            pltpu.VMEM((3, SQ, D), jnp.bfloat16),
            pltpu.VMEM((SQ, D), jnp.bfloat16),
            pltpu.VMEM((SQ, H), jnp.float32),
            pltpu.SemaphoreType.DMA((4,)),
            pltpu.SemaphoreType.DMA((4,)),
            pltpu.SemaphoreType.DMA((4,)),
            pltpu.SemaphoreType.DMA((4,)),
        ],
        compiler_params=pltpu.CompilerParams(
            collective_id=0,
            vmem_limit_bytes=44 * 1024 * 1024,
        ),
    )(xg, wq, kg, vg, wo)

    out = _ungroup(outg)
    return out.astype(jnp.float32)[None]
